# baseline (speedup 1.0000x reference)
import functools

import jax
import jax.numpy as jnp
from jax import lax
from jax.experimental import pallas as pl
from jax.experimental.pallas import tpu as pltpu
from jax.experimental.pallas import tpu_sc as plsc

U = 100000
I = 100000
D = 32
PD = 16
B = 4096
NEG = 50
K = 10
W1 = 1e-06
W2 = 1.0
W3 = 1e-06
W4 = 1.0
NEG_WEIGHT = 50.0
GAMMA = 1e-04
LAMBDA = 0.8

NC = 2
NS = 16
L = 16
NW = NC * NS
UPW = B // NW
CH = 16
HCH = CH // 2
NCHUNK = UPW // CH
IDXCH = 80
RW = 128

_C = 2048
_PSTEPS = 49
UPAD = _PSTEPS * _C

_f32 = jnp.float32
_i32 = jnp.int32


def _sel(p, off):
    r = lax.broadcasted_iota(_i32, (p, RW), 0)
    l = lax.broadcasted_iota(_i32, (p, RW), 1)
    return jnp.where(l == r + off, 1.0, 0.0).astype(_f32)


def _place(blk, off):
    return lax.dot_general(blk, _sel(blk.shape[0], off),
                           (((0,), (0,)), ((), ())),
                           preferred_element_type=_f32)


BOFF = 80


def _prep_body(uwT, fuT, iwT, fiT, iicT, iinT, bu, bi, cu_out, ci_out,
               norm_out, acc):
    i = pl.program_id(0)

    @pl.when(i == 0)
    def _init():
        acc[0] = 0.0

    colmask = (lax.broadcasted_iota(_i32, (1, _C), 1) + i * _C) < U
    uw = uwT[...]
    iw = iwT[...]
    acc[0] += (jnp.sum(jnp.where(colmask, uw * uw, 0.0))
               + jnp.sum(jnp.where(colmask, iw * iw, 0.0)))

    cu_out[...] = (_place(uw, 0) + _place(fuT[...], D)
                   + _place(bu[...].reshape(1, _C), BOFF))
    ci_out[...] = (_place(iw, 0) + _place(fiT[...], D)
                   + _place(iicT[...], D + PD)
                   + _place(iinT[...].astype(_f32), D + PD + 16)
                   + _place(bi[...].reshape(1, _C), BOFF))

    @pl.when(i == pl.num_programs(0) - 1)
    def _finish():
        norm_out[0] = acc[0]


def _prep(uwT, fuT, iwT, fiT, iicT, iinT, bu, bi):
    return pl.pallas_call(
        _prep_body,
        grid=(_PSTEPS,),
        in_specs=[
            pl.BlockSpec((D, _C), lambda i: (0, i)),
            pl.BlockSpec((PD, _C), lambda i: (0, i)),
            pl.BlockSpec((D, _C), lambda i: (0, i)),
            pl.BlockSpec((PD, _C), lambda i: (0, i)),
            pl.BlockSpec((K, _C), lambda i: (0, i)),
            pl.BlockSpec((K, _C), lambda i: (0, i)),
            pl.BlockSpec((_C,), lambda i: (i,)),
            pl.BlockSpec((_C,), lambda i: (i,)),
        ],
        out_shape=[
            jax.ShapeDtypeStruct((UPAD, RW), _f32),
            jax.ShapeDtypeStruct((UPAD, RW), _f32),
            jax.ShapeDtypeStruct((1,), _f32),
        ],
        out_specs=[
            pl.BlockSpec((_C, RW), lambda i: (i, 0)),
            pl.BlockSpec((_C, RW), lambda i: (i, 0)),
            pl.BlockSpec(memory_space=pltpu.SMEM),
        ],
        scratch_shapes=[pltpu.SMEM((1,), _f32)],
    )(uwT, fuT, iwT, fiT, iicT, iinT, bu, bi)


def _sc_body(cu, ci, users, pos, neg,
             pos_s, pos_wt, neg_s, neg_wt, nbr_s, sim_o,
             users_v, pos_v, neg_iv, nbr_iv,
             uw_v, pw_v,
             nw_a, nw_b, qw_a, qw_b,
             pos_sv, pos_wv, neg_sv, neg_wv, nbr_sv, sim_v,
             sem1, sem2, sem3, sem4, sem5):
    wid = lax.axis_index("s") * NC + lax.axis_index("c")
    lanes = lax.iota(_i32, L)
    lanek = lanes < K
    HR = HCH * NEG

    def _msum(M):
        cols = [plsc.load_gather(M, [lanes, jnp.full((L,), d_, _i32)])
                for d_ in range(L)]
        while len(cols) > 1:
            cols = [a + b for a, b in zip(cols[::2], cols[1::2])]
        return cols[0]

    def _uvec(u):
        return (uw_v[u, pl.ds(0, L)], uw_v[u, pl.ds(L, L)],
                uw_v[u, pl.ds(2 * L, L)])

    lane15 = lanes == 15

    def neg_scores(u, nw_v):
        ua, ub, uc = _uvec(u)
        uh = u % HCH
        for k in range(NEG):
            r = uh * NEG + k
            prod = (ua * nw_v[r, pl.ds(0, L)]
                    + ub * nw_v[r, pl.ds(L, L)]
                    + uc * nw_v[r, pl.ds(2 * L, L)])
            plsc.store_scatter(neg_sv, [jnp.full((L,), u * NEG + k, _i32)],
                               plsc.cumsum(prod), mask=lane15)

    def nbr_scores(u, qw_v):
        ua, ub, uc = _uvec(u)
        uh = u % (CH // 4)
        for j in range(K):
            r = uh * K + j
            prod = (ua * qw_v[r, pl.ds(0, L)]
                    + ub * qw_v[r, pl.ds(L, L)]
                    + uc * qw_v[r, pl.ds(2 * L, L)])
            plsc.store_scatter(nbr_sv, [jnp.full((L,), u * K + j, _i32)],
                               plsc.cumsum(prod), mask=lane15)

    def chunk_body(c, carry):
        base = wid * UPW + c * CH
        icps = [pltpu.async_copy(users.at[pl.ds(base, CH)], users_v, sem1),
                pltpu.async_copy(pos.at[pl.ds(base, CH)], pos_v, sem1),
                pltpu.async_copy(neg.at[pl.ds(base * NEG, CH * NEG)],
                                 neg_iv, sem1)]
        for cp in icps:
            cp.wait()

        cps = [
            pltpu.async_copy(cu.at[users_v], uw_v, sem1),
            pltpu.async_copy(ci.at[pos_v], pw_v, sem1),
        ]
        cps_h = [[], []]
        for h in range(2):
            nw_v = nw_a if h == 0 else nw_b
            for t in range(HR // IDXCH):
                sl = pl.ds(h * HR + t * IDXCH, IDXCH)
                idx = neg_iv.at[sl]
                cps_h[h].append(pltpu.async_copy(
                    ci.at[idx], nw_v.at[pl.ds(t * IDXCH, IDXCH)],
                    sem2 if h == 0 else sem3))
        for cp in cps:
            cp.wait()

        boff = jnp.full((L,), BOFF, _i32)
        pos_wv[...] = (W1 + W2
                       * plsc.load_gather(uw_v, [lanes, boff])
                       * plsc.load_gather(pw_v, [lanes, boff]))

        def neg_weights(h, nw_v):
            @plsc.parallel_loop(0, HR // L, 1, unroll=2)
            def _w(g):
                rows = g * L + lanes
                u_vec = (h * HR + rows) // NEG
                buv = plsc.load_gather(uw_v, [u_vec, boff])
                biv = plsc.load_gather(nw_v, [rows, boff])
                neg_wv[pl.ds(h * HR + g * L, L)] = W3 + W4 * buv * biv

        @plsc.parallel_loop(0, CH, 1, unroll=2)
        def user_body(u):
            ua, ub, uc = _uvec(u)
            prod = (ua * pw_v[u, pl.ds(0, L)]
                    + ub * pw_v[u, pl.ds(L, L)]
                    + uc * pw_v[u, pl.ds(2 * L, L)])
            plsc.store_scatter(pos_sv, [jnp.full((L,), u, _i32)],
                               plsc.cumsum(prod), mask=lane15)
            plsc.store_scatter(sim_v, [u * K + lanes],
                               pw_v[u, pl.ds(3 * L, L)], mask=lanek)
            plsc.store_scatter(nbr_iv, [u * K + lanes],
                               pw_v[u, pl.ds(4 * L, L)].astype(_i32),
                               mask=lanek)

        QR = (CH // 4) * K

        def fire_nbr(q):
            buf = qw_a if q % 2 == 0 else qw_b
            sem = sem4 if q % 2 == 0 else sem5
            return pltpu.async_copy(ci.at[nbr_iv.at[pl.ds(q * QR, QR)]],
                                    buf, sem)

        nbr_cps = [fire_nbr(0), fire_nbr(1)]

        for cp in cps_h[0]:
            cp.wait()
        neg_weights(0, nw_a)
        plsc.parallel_loop(0, HCH, 1, unroll=2)(
            lambda u: neg_scores(u, nw_a))
        for cp in cps_h[1]:
            cp.wait()
        neg_weights(1, nw_b)
        plsc.parallel_loop(HCH, CH, 1, unroll=2)(
            lambda u: neg_scores(u, nw_b))

        nbr_cps[0].wait()
        plsc.parallel_loop(0, 4, 1, unroll=2)(lambda u: nbr_scores(u, qw_a))
        nbr_cps.append(fire_nbr(2))
        nbr_cps[1].wait()
        plsc.parallel_loop(4, 8, 1, unroll=2)(lambda u: nbr_scores(u, qw_b))
        nbr_cps.append(fire_nbr(3))
        nbr_cps[2].wait()
        plsc.parallel_loop(8, 12, 1, unroll=2)(lambda u: nbr_scores(u, qw_a))
        nbr_cps[3].wait()
        plsc.parallel_loop(12, 16, 1, unroll=2)(lambda u: nbr_scores(u, qw_b))

        ocps = [
            pltpu.async_copy(pos_sv, pos_s.at[pl.ds(base, CH)], sem1),
            pltpu.async_copy(pos_wv, pos_wt.at[pl.ds(base, CH)], sem1),
            pltpu.async_copy(neg_sv, neg_s.at[pl.ds(base * NEG, CH * NEG)],
                             sem1),
            pltpu.async_copy(neg_wv, neg_wt.at[pl.ds(base * NEG, CH * NEG)],
                             sem1),
            pltpu.async_copy(nbr_sv, nbr_s.at[pl.ds(base * K, CH * K)], sem1),
            pltpu.async_copy(sim_v, sim_o.at[pl.ds(base * K, CH * K)], sem1),
        ]
        for cp in ocps:
            cp.wait()
        return carry

    lax.fori_loop(0, NCHUNK, chunk_body, 0, unroll=False)


_sc_call = functools.partial(
    pl.kernel,
    out_type=[
        jax.ShapeDtypeStruct((B,), _f32),
        jax.ShapeDtypeStruct((B,), _f32),
        jax.ShapeDtypeStruct((B * NEG,), _f32),
        jax.ShapeDtypeStruct((B * NEG,), _f32),
        jax.ShapeDtypeStruct((B * K,), _f32),
        jax.ShapeDtypeStruct((B * K,), _f32),
    ],
    mesh=plsc.VectorSubcoreMesh(core_axis_name="c", subcore_axis_name="s",
                                num_cores=NC, num_subcores=NS),
    compiler_params=pltpu.CompilerParams(needs_layout_passes=False,
                                         use_tc_tiling_on_sc=True),
    scratch_types=[
        pltpu.VMEM((CH,), _i32),
        pltpu.VMEM((CH,), _i32),
        pltpu.VMEM((CH * NEG,), _i32),
        pltpu.VMEM((CH * K,), _i32),
        pltpu.VMEM((CH, RW), _f32),
        pltpu.VMEM((CH, RW), _f32),
        pltpu.VMEM((CH * NEG // 2, RW), _f32),
        pltpu.VMEM((CH * NEG // 2, RW), _f32),
        pltpu.VMEM((CH // 4 * K, RW), _f32),
        pltpu.VMEM((CH // 4 * K, RW), _f32),
        pltpu.VMEM((CH,), _f32),
        pltpu.VMEM((CH,), _f32),
        pltpu.VMEM((CH * NEG,), _f32),
        pltpu.VMEM((CH * NEG,), _f32),
        pltpu.VMEM((CH * K,), _f32),
        pltpu.VMEM((CH * K,), _f32),
        pltpu.SemaphoreType.DMA,
        pltpu.SemaphoreType.DMA,
        pltpu.SemaphoreType.DMA,
        pltpu.SemaphoreType.DMA,
        pltpu.SemaphoreType.DMA,
    ],
)(_sc_body)


def _fin_body(ps, pw, ns, nw, qs, sm, nrm, out):
    pos_part = jnp.sum(pw[...] * jax.nn.softplus(-ps[...]))
    neg_part = jnp.sum(nw[...] * jax.nn.softplus(ns[...])) * (NEG_WEIGHT / NEG)
    nbr_part = jnp.sum(sm[...] * jax.nn.softplus(-qs[...]))
    out[0] = (pos_part + neg_part + LAMBDA * nbr_part
              + (0.5 * GAMMA) * nrm[0])


def kernel(user_w, item_w, frozen_u, frozen_i, beta_uD, beta_iD,
           ii_constraint, ii_neighbor, users, pos_items, neg_items):
    users = users.astype(_i32)
    pos = pos_items.astype(_i32)
    neg_flat = neg_items.reshape(-1).astype(_i32)

    cu, ci, norm = _prep(user_w.T, frozen_u.T, item_w.T, frozen_i.T,
                         ii_constraint.T, ii_neighbor.astype(_i32).T,
                         beta_uD, beta_iD)

    pos_s, pos_wt, neg_s, neg_wt, nbr_s, sim = _sc_call(
        cu, ci, users, pos, neg_flat)

    total = pl.pallas_call(
        _fin_body,
        out_shape=jax.ShapeDtypeStruct((1,), _f32),
        in_specs=[
            pl.BlockSpec((B // 128, 128), lambda: (0, 0)),
            pl.BlockSpec((B // 128, 128), lambda: (0, 0)),
            pl.BlockSpec((B * NEG // 128, 128), lambda: (0, 0)),
            pl.BlockSpec((B * NEG // 128, 128), lambda: (0, 0)),
            pl.BlockSpec((B * K // 128, 128), lambda: (0, 0)),
            pl.BlockSpec((B * K // 128, 128), lambda: (0, 0)),
            pl.BlockSpec(memory_space=pltpu.SMEM),
        ],
        out_specs=pl.BlockSpec(memory_space=pltpu.SMEM),
    )(pos_s.reshape(B // 128, 128), pos_wt.reshape(B // 128, 128),
      neg_s.reshape(B * NEG // 128, 128), neg_wt.reshape(B * NEG // 128, 128),
      nbr_s.reshape(B * K // 128, 128), sim.reshape(B * K // 128, 128),
      norm)
    return total[0]


# speedup vs baseline: 1.1310x; 1.1310x over previous
import functools

import jax
import jax.numpy as jnp
from jax import lax
from jax.experimental import pallas as pl
from jax.experimental.pallas import tpu as pltpu
from jax.experimental.pallas import tpu_sc as plsc

U = 100000
I = 100000
D = 32
PD = 16
B = 4096
NEG = 50
K = 10
W1 = 1e-06
W2 = 1.0
W3 = 1e-06
W4 = 1.0
NEG_WEIGHT = 50.0
GAMMA = 1e-04
LAMBDA = 0.8

NC = 2
NS = 16
L = 16
NW = NC * NS
UPW = B // NW
CH = 16
HCH = CH // 2
NCHUNK = UPW // CH
IDXCH = 80
RW = 128

_C = 2048
_PSTEPS = 49
UPAD = _PSTEPS * _C

_f32 = jnp.float32
_i32 = jnp.int32


def _sel(p, off):
    r = lax.broadcasted_iota(_i32, (p, RW), 0)
    l = lax.broadcasted_iota(_i32, (p, RW), 1)
    return jnp.where(l == r + off, 1.0, 0.0).astype(_f32)


def _place(blk, off):
    return lax.dot_general(blk, _sel(blk.shape[0], off),
                           (((0,), (0,)), ((), ())),
                           preferred_element_type=_f32)


def _prep_body(uwT, fuT, iwT, fiT, iicT, iinT, cu_out, ci_out,
               norm_out, acc):
    i = pl.program_id(0)

    @pl.when(i == 0)
    def _init():
        acc[0] = 0.0

    colmask = (lax.broadcasted_iota(_i32, (1, _C), 1) + i * _C) < U
    uw = uwT[...]
    iw = iwT[...]
    acc[0] += (jnp.sum(jnp.where(colmask, uw * uw, 0.0))
               + jnp.sum(jnp.where(colmask, iw * iw, 0.0)))

    cu_out[...] = _place(uw, 0) + _place(fuT[...], D)
    ci_out[...] = (_place(iw, 0) + _place(fiT[...], D)
                   + _place(iicT[...], D + PD)
                   + _place(iinT[...].astype(_f32), D + PD + 16))

    @pl.when(i == pl.num_programs(0) - 1)
    def _finish():
        norm_out[0] = acc[0]


def _prep(uwT, fuT, iwT, fiT, iicT, iinT):
    return pl.pallas_call(
        _prep_body,
        grid=(_PSTEPS,),
        in_specs=[
            pl.BlockSpec((D, _C), lambda i: (0, i)),
            pl.BlockSpec((PD, _C), lambda i: (0, i)),
            pl.BlockSpec((D, _C), lambda i: (0, i)),
            pl.BlockSpec((PD, _C), lambda i: (0, i)),
            pl.BlockSpec((K, _C), lambda i: (0, i)),
            pl.BlockSpec((K, _C), lambda i: (0, i)),
        ],
        out_shape=[
            jax.ShapeDtypeStruct((UPAD, RW), _f32),
            jax.ShapeDtypeStruct((UPAD, RW), _f32),
            jax.ShapeDtypeStruct((1,), _f32),
        ],
        out_specs=[
            pl.BlockSpec((_C, RW), lambda i: (i, 0)),
            pl.BlockSpec((_C, RW), lambda i: (i, 0)),
            pl.BlockSpec(memory_space=pltpu.SMEM),
        ],
        scratch_shapes=[pltpu.SMEM((1,), _f32)],
    )(uwT, fuT, iwT, fiT, iicT, iinT)


def _sc_body(cu, ci, beta_u, beta_i, users, pos, neg,
             pos_s, pos_wt, neg_s, neg_wt, nbr_s, sim_o,
             users_v, pos_v, neg_iv, nbr_iv,
             uw_v, pw_v, bu_v, bip_v, bin_v,
             nw_a, nw_b, qw_a, qw_b,
             pos_sv, pos_wv, neg_sv, neg_wv, nbr_sv, sim_v,
             sem1, sem2, sem3, sem4, sem5):
    wid = lax.axis_index("s") * NC + lax.axis_index("c")
    lanes = lax.iota(_i32, L)
    lanek = lanes < K
    HR = HCH * NEG

    def _msum(M):
        cols = [plsc.load_gather(M, [lanes, jnp.full((L,), d_, _i32)])
                for d_ in range(L)]
        while len(cols) > 1:
            cols = [a + b for a, b in zip(cols[::2], cols[1::2])]
        return cols[0]

    def _uvec(u):
        return (uw_v[u, pl.ds(0, L)], uw_v[u, pl.ds(L, L)],
                uw_v[u, pl.ds(2 * L, L)])

    lane15 = lanes == 15

    def neg_scores(u, nw_v):
        ua, ub, uc = _uvec(u)
        uh = u % HCH
        for k in range(NEG):
            r = uh * NEG + k
            prod = (ua * nw_v[r, pl.ds(0, L)]
                    + ub * nw_v[r, pl.ds(L, L)]
                    + uc * nw_v[r, pl.ds(2 * L, L)])
            plsc.store_scatter(neg_sv, [jnp.full((L,), u * NEG + k, _i32)],
                               plsc.cumsum(prod), mask=lane15)

    def nbr_scores(u, qw_v):
        ua, ub, uc = _uvec(u)
        uh = u % (CH // 4)
        for j in range(K):
            r = uh * K + j
            prod = (ua * qw_v[r, pl.ds(0, L)]
                    + ub * qw_v[r, pl.ds(L, L)]
                    + uc * qw_v[r, pl.ds(2 * L, L)])
            plsc.store_scatter(nbr_sv, [jnp.full((L,), u * K + j, _i32)],
                               plsc.cumsum(prod), mask=lane15)

    def chunk_body(c, carry):
        base = wid * UPW + c * CH
        icps = [pltpu.async_copy(users.at[pl.ds(base, CH)], users_v, sem1),
                pltpu.async_copy(pos.at[pl.ds(base, CH)], pos_v, sem1),
                pltpu.async_copy(neg.at[pl.ds(base * NEG, CH * NEG)],
                                 neg_iv, sem1)]
        for cp in icps:
            cp.wait()

        cps = [
            pltpu.async_copy(cu.at[users_v], uw_v, sem1),
            pltpu.async_copy(ci.at[pos_v], pw_v, sem1),
            pltpu.async_copy(beta_u.at[users_v], bu_v, sem1),
            pltpu.async_copy(beta_i.at[pos_v], bip_v, sem1),
        ]
        cps_h = [[], []]
        for h in range(2):
            nw_v = nw_a if h == 0 else nw_b
            for t in range(HR // IDXCH):
                sl = pl.ds(h * HR + t * IDXCH, IDXCH)
                idx = neg_iv.at[sl]
                cps_h[h].append(pltpu.async_copy(
                    ci.at[idx], nw_v.at[pl.ds(t * IDXCH, IDXCH)],
                    sem2 if h == 0 else sem3))
                cps.append(pltpu.async_copy(beta_i.at[idx], bin_v.at[sl],
                                            sem1))
        for cp in cps:
            cp.wait()

        @plsc.parallel_loop(0, CH * NEG // L, 1, unroll=2)
        def _weights(g):
            r0 = g * L
            u_vec = (lanes + r0) // NEG
            buv = plsc.load_gather(bu_v, [u_vec])
            neg_wv[pl.ds(r0, L)] = W3 + W4 * buv * bin_v[pl.ds(r0, L)]

        pos_wv[...] = W1 + W2 * bu_v[...] * bip_v[...]

        @plsc.parallel_loop(0, CH, 1, unroll=2)
        def user_body(u):
            ua, ub, uc = _uvec(u)
            prod = (ua * pw_v[u, pl.ds(0, L)]
                    + ub * pw_v[u, pl.ds(L, L)]
                    + uc * pw_v[u, pl.ds(2 * L, L)])
            plsc.store_scatter(pos_sv, [jnp.full((L,), u, _i32)],
                               plsc.cumsum(prod), mask=lane15)
            plsc.store_scatter(sim_v, [u * K + lanes],
                               pw_v[u, pl.ds(3 * L, L)], mask=lanek)
            plsc.store_scatter(nbr_iv, [u * K + lanes],
                               pw_v[u, pl.ds(4 * L, L)].astype(_i32),
                               mask=lanek)

        QR = (CH // 4) * K

        def fire_nbr(q):
            buf = qw_a if q % 2 == 0 else qw_b
            sem = sem4 if q % 2 == 0 else sem5
            return pltpu.async_copy(ci.at[nbr_iv.at[pl.ds(q * QR, QR)]],
                                    buf, sem)

        nbr_cps = [fire_nbr(0), fire_nbr(1)]

        for cp in cps_h[0]:
            cp.wait()
        plsc.parallel_loop(0, HCH, 1, unroll=2)(
            lambda u: neg_scores(u, nw_a))
        for cp in cps_h[1]:
            cp.wait()
        plsc.parallel_loop(HCH, CH, 1, unroll=2)(
            lambda u: neg_scores(u, nw_b))

        nbr_cps[0].wait()
        plsc.parallel_loop(0, 4, 1, unroll=2)(lambda u: nbr_scores(u, qw_a))
        nbr_cps.append(fire_nbr(2))
        nbr_cps[1].wait()
        plsc.parallel_loop(4, 8, 1, unroll=2)(lambda u: nbr_scores(u, qw_b))
        nbr_cps.append(fire_nbr(3))
        nbr_cps[2].wait()
        plsc.parallel_loop(8, 12, 1, unroll=2)(lambda u: nbr_scores(u, qw_a))
        nbr_cps[3].wait()
        plsc.parallel_loop(12, 16, 1, unroll=2)(lambda u: nbr_scores(u, qw_b))

        ocps = [
            pltpu.async_copy(pos_sv, pos_s.at[pl.ds(base, CH)], sem1),
            pltpu.async_copy(pos_wv, pos_wt.at[pl.ds(base, CH)], sem1),
            pltpu.async_copy(neg_sv, neg_s.at[pl.ds(base * NEG, CH * NEG)],
                             sem1),
            pltpu.async_copy(neg_wv, neg_wt.at[pl.ds(base * NEG, CH * NEG)],
                             sem1),
            pltpu.async_copy(nbr_sv, nbr_s.at[pl.ds(base * K, CH * K)], sem1),
            pltpu.async_copy(sim_v, sim_o.at[pl.ds(base * K, CH * K)], sem1),
        ]
        for cp in ocps:
            cp.wait()
        return carry

    lax.fori_loop(0, NCHUNK, chunk_body, 0, unroll=False)


_sc_call = functools.partial(
    pl.kernel,
    out_type=[
        jax.ShapeDtypeStruct((B,), _f32),
        jax.ShapeDtypeStruct((B,), _f32),
        jax.ShapeDtypeStruct((B * NEG,), _f32),
        jax.ShapeDtypeStruct((B * NEG,), _f32),
        jax.ShapeDtypeStruct((B * K,), _f32),
        jax.ShapeDtypeStruct((B * K,), _f32),
    ],
    mesh=plsc.VectorSubcoreMesh(core_axis_name="c", subcore_axis_name="s",
                                num_cores=NC, num_subcores=NS),
    compiler_params=pltpu.CompilerParams(needs_layout_passes=False,
                                         use_tc_tiling_on_sc=True),
    scratch_types=[
        pltpu.VMEM((CH,), _i32),
        pltpu.VMEM((CH,), _i32),
        pltpu.VMEM((CH * NEG,), _i32),
        pltpu.VMEM((CH * K,), _i32),
        pltpu.VMEM((CH, RW), _f32),
        pltpu.VMEM((CH, RW), _f32),
        pltpu.VMEM((CH,), _f32),
        pltpu.VMEM((CH,), _f32),
        pltpu.VMEM((CH * NEG,), _f32),
        pltpu.VMEM((CH * NEG // 2, RW), _f32),
        pltpu.VMEM((CH * NEG // 2, RW), _f32),
        pltpu.VMEM((CH // 4 * K, RW), _f32),
        pltpu.VMEM((CH // 4 * K, RW), _f32),
        pltpu.VMEM((CH,), _f32),
        pltpu.VMEM((CH,), _f32),
        pltpu.VMEM((CH * NEG,), _f32),
        pltpu.VMEM((CH * NEG,), _f32),
        pltpu.VMEM((CH * K,), _f32),
        pltpu.VMEM((CH * K,), _f32),
        pltpu.SemaphoreType.DMA,
        pltpu.SemaphoreType.DMA,
        pltpu.SemaphoreType.DMA,
        pltpu.SemaphoreType.DMA,
        pltpu.SemaphoreType.DMA,
    ],
)(_sc_body)


def _fin_body(ps, pw, ns, nw, qs, sm, nrm, out):
    pos_part = jnp.sum(pw[...] * jax.nn.softplus(-ps[...]))
    neg_part = jnp.sum(nw[...] * jax.nn.softplus(ns[...])) * (NEG_WEIGHT / NEG)
    nbr_part = jnp.sum(sm[...] * jax.nn.softplus(-qs[...]))
    out[0] = (pos_part + neg_part + LAMBDA * nbr_part
              + (0.5 * GAMMA) * nrm[0])


def kernel(user_w, item_w, frozen_u, frozen_i, beta_uD, beta_iD,
           ii_constraint, ii_neighbor, users, pos_items, neg_items):
    users = users.astype(_i32)
    pos = pos_items.astype(_i32)
    neg_flat = neg_items.reshape(-1).astype(_i32)

    cu, ci, norm = _prep(user_w.T, frozen_u.T, item_w.T, frozen_i.T,
                         ii_constraint.T, ii_neighbor.astype(_i32).T)

    pos_s, pos_wt, neg_s, neg_wt, nbr_s, sim = _sc_call(
        cu, ci, beta_uD, beta_iD, users, pos, neg_flat)

    total = pl.pallas_call(
        _fin_body,
        out_shape=jax.ShapeDtypeStruct((1,), _f32),
        in_specs=[
            pl.BlockSpec((B // 128, 128), lambda: (0, 0)),
            pl.BlockSpec((B // 128, 128), lambda: (0, 0)),
            pl.BlockSpec((B * NEG // 128, 128), lambda: (0, 0)),
            pl.BlockSpec((B * NEG // 128, 128), lambda: (0, 0)),
            pl.BlockSpec((B * K // 128, 128), lambda: (0, 0)),
            pl.BlockSpec((B * K // 128, 128), lambda: (0, 0)),
            pl.BlockSpec(memory_space=pltpu.SMEM),
        ],
        out_specs=pl.BlockSpec(memory_space=pltpu.SMEM),
    )(pos_s.reshape(B // 128, 128), pos_wt.reshape(B // 128, 128),
      neg_s.reshape(B * NEG // 128, 128), neg_wt.reshape(B * NEG // 128, 128),
      nbr_s.reshape(B * K // 128, 128), sim.reshape(B * K // 128, 128),
      norm)
    return total[0]
